# baseline (speedup 1.0000x reference)
import dataclasses

import jax
import jax.numpy as jnp
from jax import lax
from jax.experimental import pallas as pl
from jax.experimental.pallas import tpu as pltpu
from jax.experimental.pallas import tpu_sc as plsc

B = 1024
IN_F = 4096
OUT_F = 4096
K = 16
LANES = 16

BC = 16
NB = 1024
N_CHUNKS = B // BC
NUM_WORKERS = 32
CHUNKS_PER_W = N_CHUNKS // NUM_WORKERS
GROUPS = NB // LANES
NBLKS = OUT_F // NB


def _body(inp_hbm, wt_hbm, bias_hbm, maskt_hbm, out_hbm,
          inp_v, w_v, m_v, bias_v, out_v):
    c = lax.axis_index("c")
    s = lax.axis_index("s")
    wid = s * 2 + c

    pltpu.sync_copy(bias_hbm, bias_v)

    def chunk_body(ci, carry):
        b0 = (wid * CHUNKS_PER_W + ci) * BC
        pltpu.sync_copy(inp_hbm.at[pl.ds(b0, BC), :], inp_v)

        def nb_body(nb, carry2):
            n0 = nb * NB
            pltpu.sync_copy(wt_hbm.at[:, pl.ds(n0, NB)], w_v)
            pltpu.sync_copy(maskt_hbm.at[:, pl.ds(n0, NB)], m_v)

            @plsc.parallel_loop(0, GROUPS, unroll=2)
            def g_body(g):
                gs = g * LANES
                bias_vec = bias_v[pl.ds(n0 + gs, LANES)]
                for bh in range(4):
                    accs = [bias_vec] * (BC // 4)
                    for k in range(K):
                        mk = m_v[k, pl.ds(gs, LANES)]
                        wk = w_v[k, pl.ds(gs, LANES)]
                        for bj in range(BC // 4):
                            b = bh * (BC // 4) + bj
                            bvec = jnp.full((LANES,), b, jnp.int32)
                            gat = plsc.load_gather(inp_v, [bvec, mk])
                            accs[bj] = accs[bj] + gat * wk
                    for bj in range(BC // 4):
                        b = bh * (BC // 4) + bj
                        out_v[b, pl.ds(gs, LANES)] = accs[bj]

            pltpu.sync_copy(out_v, out_hbm.at[pl.ds(b0, BC), pl.ds(n0, NB)])
            return carry2

        lax.fori_loop(0, NBLKS, nb_body, 0)
        return carry

    lax.fori_loop(0, CHUNKS_PER_W, chunk_body, 0)


@jax.jit
def kernel(input, condensed_weight, bias, input_mask):
    wt = condensed_weight.T
    maskt = input_mask.T.astype(jnp.int32)
    mesh = plsc.VectorSubcoreMesh(core_axis_name="c", subcore_axis_name="s")
    cp = pltpu.CompilerParams()
    if "needs_layout_passes" in pltpu.CompilerParams.__dataclass_fields__:
        cp = dataclasses.replace(cp, needs_layout_passes=False)
    f = pl.kernel(
        _body,
        out_type=jax.ShapeDtypeStruct((B, OUT_F), jnp.float32),
        mesh=mesh,
        scratch_types=[
            pltpu.VMEM((BC, IN_F), jnp.float32),
            pltpu.VMEM((K, NB), jnp.float32),
            pltpu.VMEM((K, NB), jnp.int32),
            pltpu.VMEM((OUT_F,), jnp.float32),
            pltpu.VMEM((BC, NB), jnp.float32),
        ],
        compiler_params=cp,
    )
    return f(input, wt, bias, maskt)


# speedup vs baseline: 1.0146x; 1.0146x over previous
import dataclasses

import jax
import jax.numpy as jnp
from jax import lax
from jax.experimental import pallas as pl
from jax.experimental.pallas import tpu as pltpu
from jax.experimental.pallas import tpu_sc as plsc

B = 1024
IN_F = 4096
OUT_F = 4096
K = 16
LANES = 16

BC = 16
NB = 1024
N_CHUNKS = B // BC
NUM_WORKERS = 32
CHUNKS_PER_W = N_CHUNKS // NUM_WORKERS
GROUPS = NB // LANES
NBLKS = OUT_F // NB


def _body(inp_hbm, wt_hbm, bias_hbm, maskt_hbm, out_hbm,
          inp_v, w_v, m_v, bias_v, out_v):
    c = lax.axis_index("c")
    s = lax.axis_index("s")
    wid = s * 2 + c

    pltpu.sync_copy(bias_hbm, bias_v)

    def chunk_body(ci, carry):
        b0 = (wid * CHUNKS_PER_W + ci) * BC
        pltpu.sync_copy(inp_hbm.at[pl.ds(b0, BC), :], inp_v)

        def nb_body(nb, carry2):
            n0 = nb * NB
            pltpu.sync_copy(wt_hbm.at[:, pl.ds(n0, NB)], w_v)
            pltpu.sync_copy(maskt_hbm.at[:, pl.ds(n0, NB)], m_v)

            @plsc.parallel_loop(0, GROUPS, unroll=2)
            def g_body(g):
                gs = g * LANES
                bias_vec = bias_v[pl.ds(n0 + gs, LANES)]
                for bh in range(2):
                    accs = [bias_vec] * (BC // 2)
                    for k in range(K):
                        mk = m_v[k, pl.ds(gs, LANES)]
                        wk = w_v[k, pl.ds(gs, LANES)]
                        for bj in range(BC // 2):
                            b = bh * (BC // 2) + bj
                            bvec = jnp.full((LANES,), b, jnp.int32)
                            gat = plsc.load_gather(inp_v, [bvec, mk])
                            accs[bj] = accs[bj] + gat * wk
                    for bj in range(BC // 2):
                        b = bh * (BC // 2) + bj
                        out_v[b, pl.ds(gs, LANES)] = accs[bj]

            pltpu.sync_copy(out_v, out_hbm.at[pl.ds(b0, BC), pl.ds(n0, NB)])
            return carry2

        lax.fori_loop(0, NBLKS, nb_body, 0)
        return carry

    lax.fori_loop(0, CHUNKS_PER_W, chunk_body, 0)


@jax.jit
def kernel(input, condensed_weight, bias, input_mask):
    wt = condensed_weight.T
    maskt = input_mask.T.astype(jnp.int32)
    mesh = plsc.VectorSubcoreMesh(core_axis_name="c", subcore_axis_name="s")
    cp = pltpu.CompilerParams()
    if "needs_layout_passes" in pltpu.CompilerParams.__dataclass_fields__:
        cp = dataclasses.replace(cp, needs_layout_passes=False)
    f = pl.kernel(
        _body,
        out_type=jax.ShapeDtypeStruct((B, OUT_F), jnp.float32),
        mesh=mesh,
        scratch_types=[
            pltpu.VMEM((BC, IN_F), jnp.float32),
            pltpu.VMEM((K, NB), jnp.float32),
            pltpu.VMEM((K, NB), jnp.int32),
            pltpu.VMEM((OUT_F,), jnp.float32),
            pltpu.VMEM((BC, NB), jnp.float32),
        ],
        compiler_params=cp,
    )
    return f(input, wt, bias, maskt)


# speedup vs baseline: 1.5654x; 1.5428x over previous
import dataclasses
import functools

import jax
import jax.numpy as jnp
from jax import lax
from jax.experimental import pallas as pl
from jax.experimental.pallas import tpu as pltpu
from jax.experimental.pallas import tpu_sc as plsc

B = 1024
IN_F = 4096
OUT_F = 4096
K = 16
LANES = 16

N_SC = 2048
N_TC = OUT_F - N_SC

BC = 16
NB = 1024
N_CHUNKS = B // BC
NUM_WORKERS = 32
CHUNKS_PER_W = N_CHUNKS // NUM_WORKERS
GROUPS = NB // LANES
NBLKS = N_SC // NB

NTB = 256


def _sc_body(inp_hbm, wt_hbm, bias_hbm, maskt_hbm, out_hbm,
             inp_v, w_v, m_v, bias_v, out_v):
    c = lax.axis_index("c")
    s = lax.axis_index("s")
    wid = s * 2 + c

    pltpu.sync_copy(bias_hbm, bias_v)

    def chunk_body(ci, carry):
        b0 = (wid * CHUNKS_PER_W + ci) * BC
        pltpu.sync_copy(inp_hbm.at[pl.ds(b0, BC), :], inp_v)

        def nb_body(nb, carry2):
            n0 = nb * NB
            pltpu.sync_copy(wt_hbm.at[:, pl.ds(n0, NB)], w_v)
            pltpu.sync_copy(maskt_hbm.at[:, pl.ds(n0, NB)], m_v)

            @plsc.parallel_loop(0, GROUPS, unroll=2)
            def g_body(g):
                gs = g * LANES
                bias_vec = bias_v[pl.ds(n0 + gs, LANES)]
                for bh in range(2):
                    accs = [bias_vec] * (BC // 2)
                    for k in range(K):
                        mk = m_v[k, pl.ds(gs, LANES)]
                        wk = w_v[k, pl.ds(gs, LANES)]
                        for bj in range(BC // 2):
                            b = bh * (BC // 2) + bj
                            bvec = jnp.full((LANES,), b, jnp.int32)
                            gat = plsc.load_gather(inp_v, [bvec, mk])
                            accs[bj] = accs[bj] + gat * wk
                    for bj in range(BC // 2):
                        b = bh * (BC // 2) + bj
                        out_v[b, pl.ds(gs, LANES)] = accs[bj]

            pltpu.sync_copy(out_v, out_hbm.at[pl.ds(b0, BC), pl.ds(n0, NB)])
            return carry2

        lax.fori_loop(0, NBLKS, nb_body, 0)
        return carry

    lax.fori_loop(0, CHUNKS_PER_W, chunk_body, 0)


def _tc_body(inp_ref, mask_ref, w_ref, bias_ref, out_ref):
    f_iota = lax.broadcasted_iota(jnp.int32, (IN_F, NTB), 0)
    s_acc = jnp.zeros((IN_F, NTB), jnp.float32)
    for k in range(K):
        mk = mask_ref[:, k].reshape(1, NTB)
        wk = w_ref[:, k].reshape(1, NTB)
        s_acc = s_acc + jnp.where(f_iota == mk, wk, 0.0)
    out_ref[...] = (
        jnp.dot(inp_ref[...], s_acc.astype(jnp.bfloat16),
                preferred_element_type=jnp.float32)
        + bias_ref[...]
    )


@jax.jit
def kernel(input, condensed_weight, bias, input_mask):
    maski = input_mask.astype(jnp.int32)
    wt_sc = condensed_weight[:N_SC].T
    maskt_sc = maski[:N_SC].T

    mesh = plsc.VectorSubcoreMesh(core_axis_name="c", subcore_axis_name="s")
    cp = pltpu.CompilerParams()
    if "needs_layout_passes" in pltpu.CompilerParams.__dataclass_fields__:
        cp = dataclasses.replace(cp, needs_layout_passes=False)
    sc_fn = pl.kernel(
        _sc_body,
        out_type=jax.ShapeDtypeStruct((B, N_SC), jnp.float32),
        mesh=mesh,
        scratch_types=[
            pltpu.VMEM((BC, IN_F), jnp.float32),
            pltpu.VMEM((K, NB), jnp.float32),
            pltpu.VMEM((K, NB), jnp.int32),
            pltpu.VMEM((N_SC,), jnp.float32),
            pltpu.VMEM((BC, NB), jnp.float32),
        ],
        compiler_params=cp,
    )
    out_sc = sc_fn(input, wt_sc, bias[:N_SC], maskt_sc)

    tc_fn = pl.pallas_call(
        _tc_body,
        out_shape=jax.ShapeDtypeStruct((B, N_TC), jnp.float32),
        grid=(N_TC // NTB,),
        in_specs=[
            pl.BlockSpec((B, IN_F), lambda i: (0, 0)),
            pl.BlockSpec((NTB, K), lambda i: (i, 0)),
            pl.BlockSpec((NTB, K), lambda i: (i, 0)),
            pl.BlockSpec((1, NTB), lambda i: (0, i)),
        ],
        out_specs=pl.BlockSpec((B, NTB), lambda i: (0, i)),
    )
    out_tc = tc_fn(input.astype(jnp.bfloat16), maski[N_SC:],
                   condensed_weight[N_SC:], bias[None, N_SC:])

    return jnp.concatenate([out_sc, out_tc], axis=1)
